# baseline (speedup 1.0000x reference)
import functools

import jax
import jax.numpy as jnp
from jax import lax
from jax.experimental import pallas as pl
from jax.experimental.pallas import tpu as pltpu
from jax.experimental.pallas import tpu_sc as plsc

_LANES = 16


def _sc_pool(table, x):
    B, S = x.shape
    V, D = table.shape
    info = plsc.get_sparse_core_info()
    NC, NS = info.num_cores, info.num_subcores
    NW = NC * NS
    assert B % NW == 0
    BPW = B // NW
    CB = 4
    assert BPW % CB == 0
    NIT = BPW // CB
    assert S % 8 == 0 and D == 2 * _LANES

    mesh = plsc.VectorSubcoreMesh(core_axis_name="c", subcore_axis_name="s",
                                  num_cores=NC, num_subcores=NS)

    @functools.partial(
        pl.kernel,
        out_type=jax.ShapeDtypeStruct((B, D), jnp.float32),
        mesh=mesh,
        compiler_params=pltpu.CompilerParams(use_tc_tiling_on_sc=False),
        scratch_types=[
            pltpu.VMEM((CB, S), jnp.int32),
            pltpu.VMEM((CB * S, D), jnp.float32),
            pltpu.VMEM((BPW, D), jnp.float32),
            pltpu.SemaphoreType.DMA,
        ],
    )
    def pool(tbl_hbm, x_hbm, out_hbm, idx_v, rows_v, outb_v, sem):
        wid = lax.axis_index("s") * NC + lax.axis_index("c")
        base_b = wid * BPW
        inv = 1.0 / S

        def chunk_body(it, carry):
            row0 = base_b + it * CB
            pltpu.sync_copy(x_hbm.at[pl.ds(row0, CB)], idx_v)
            cps = [
                pltpu.async_copy(tbl_hbm.at[idx_v.at[bi]],
                                 rows_v.at[pl.ds(bi * S, S)], sem)
                for bi in range(CB)
            ]
            for cp in cps:
                cp.wait()
            for bi in range(CB):
                rb = bi * S
                for h in range(2):
                    lo = h * _LANES

                    def grp(j, acc):
                        base = rb + j * 8
                        r0 = rows_v[base + 0, pl.ds(lo, _LANES)]
                        r1 = rows_v[base + 1, pl.ds(lo, _LANES)]
                        r2 = rows_v[base + 2, pl.ds(lo, _LANES)]
                        r3 = rows_v[base + 3, pl.ds(lo, _LANES)]
                        r4 = rows_v[base + 4, pl.ds(lo, _LANES)]
                        r5 = rows_v[base + 5, pl.ds(lo, _LANES)]
                        r6 = rows_v[base + 6, pl.ds(lo, _LANES)]
                        r7 = rows_v[base + 7, pl.ds(lo, _LANES)]
                        return acc + (((r0 + r1) + (r2 + r3))
                                      + ((r4 + r5) + (r6 + r7)))

                    acc = lax.fori_loop(0, S // 8, grp,
                                        jnp.zeros((_LANES,), jnp.float32))
                    outb_v[it * CB + bi, pl.ds(lo, _LANES)] = acc * inv
            return carry

        lax.fori_loop(0, NIT, chunk_body, 0)
        pltpu.sync_copy(outb_v, out_hbm.at[pl.ds(base_b, BPW)])

    return pool(table, x)


def _cls_matmul(pooled, wpt, bvec):
    B, D = pooled.shape

    def body(p_ref, w_ref, b_ref, out_ref):
        out_ref[...] = (jnp.dot(p_ref[...], w_ref[...],
                                preferred_element_type=jnp.float32)
                        + b_ref[...])

    return pl.pallas_call(
        body,
        in_specs=[
            pl.BlockSpec((B, D), lambda: (0, 0)),
            pl.BlockSpec((D, _LANES), lambda: (0, 0)),
            pl.BlockSpec((1, _LANES), lambda: (0, 0)),
        ],
        out_specs=pl.BlockSpec((B, _LANES), lambda: (0, 0)),
        out_shape=jax.ShapeDtypeStruct((B, _LANES), jnp.float32),
    )(pooled, wpt, bvec)


def kernel(x, table, W, b):
    B, S = x.shape
    V, D = table.shape
    C = W.shape[0]
    wpt = jnp.zeros((D, _LANES), jnp.float32).at[:, :C].set(W.T)
    bvec = jnp.zeros((1, _LANES), jnp.float32).at[0, :C].set(b)
    pooled = _sc_pool(table, x.astype(jnp.int32))
    logits = _cls_matmul(pooled, wpt, bvec)
    return logits[:, :C]


# speedup vs baseline: 1.3394x; 1.3394x over previous
import functools

import jax
import jax.numpy as jnp
from jax import lax
from jax.experimental import pallas as pl
from jax.experimental.pallas import tpu as pltpu
from jax.experimental.pallas import tpu_sc as plsc

_LANES = 16


def _fmt(tablet):
    D, V = tablet.shape
    blk = 16384

    def body(t_ref, o_ref):
        eye = jnp.eye(D, 128, dtype=jnp.float32)
        o_ref[...] = lax.dot_general(
            t_ref[...], eye, (((0,), (0,)), ((), ())),
            preferred_element_type=jnp.float32)

    return pl.pallas_call(
        body,
        grid=(pl.cdiv(V, blk),),
        in_specs=[pl.BlockSpec((D, blk), lambda i: (0, i))],
        out_specs=pl.BlockSpec((blk, 128), lambda i: (i, 0)),
        out_shape=jax.ShapeDtypeStruct((V, 128), jnp.float32),
    )(tablet)


def _sc_pool(table4, x1d, B, S):
    D = table4.shape[1]
    info = plsc.get_sparse_core_info()
    NC, NS = info.num_cores, info.num_subcores
    NW = NC * NS
    assert B % NW == 0
    BPW = B // NW
    CB = 4
    assert BPW % CB == 0
    NIT = BPW // CB
    assert S % 8 == 0 and (CB * S) % _LANES == 0 and D == 2 * _LANES
    NIV = (CB * S) // _LANES

    mesh = plsc.VectorSubcoreMesh(core_axis_name="c", subcore_axis_name="s",
                                  num_cores=NC, num_subcores=NS)

    @functools.partial(
        pl.kernel,
        out_type=jax.ShapeDtypeStruct((B, D), jnp.float32),
        mesh=mesh,
        compiler_params=pltpu.CompilerParams(use_tc_tiling_on_sc=False),
        scratch_types=[
            pltpu.VMEM((CB * S,), jnp.int32),
            pltpu.VMEM((CB * S, D), jnp.float32),
            pltpu.VMEM((BPW, D), jnp.float32),
            pltpu.SemaphoreType.DMA,
        ],
    )
    def pool(tbl_hbm, x_hbm, out_hbm, idx_v, rows_v, outb_v, sem):
        wid = lax.axis_index("s") * NC + lax.axis_index("c")
        base_b = wid * BPW
        inv = 1.0 / S

        def chunk_body(it, carry):
            row0 = base_b + it * CB
            pltpu.sync_copy(x_hbm.at[pl.ds(row0 * S, CB * S)], idx_v)

            def xform(k, carry2):
                u = idx_v[pl.ds(k * _LANES, _LANES)]
                idx_v[pl.ds(k * _LANES, _LANES)] = u * 4
                return carry2

            lax.fori_loop(0, NIV, xform, 0)
            cps = [
                pltpu.async_copy(tbl_hbm.at[idx_v.at[pl.ds(bi * S, S)]],
                                 rows_v.at[pl.ds(bi * S, S)], sem)
                for bi in range(CB)
            ]
            for cp in cps:
                cp.wait()
            for bi in range(CB):
                rb = bi * S
                for h in range(2):
                    lo = h * _LANES

                    def grp(j, acc):
                        base = rb + j * 8
                        r0 = rows_v[base + 0, pl.ds(lo, _LANES)]
                        r1 = rows_v[base + 1, pl.ds(lo, _LANES)]
                        r2 = rows_v[base + 2, pl.ds(lo, _LANES)]
                        r3 = rows_v[base + 3, pl.ds(lo, _LANES)]
                        r4 = rows_v[base + 4, pl.ds(lo, _LANES)]
                        r5 = rows_v[base + 5, pl.ds(lo, _LANES)]
                        r6 = rows_v[base + 6, pl.ds(lo, _LANES)]
                        r7 = rows_v[base + 7, pl.ds(lo, _LANES)]
                        return acc + (((r0 + r1) + (r2 + r3))
                                      + ((r4 + r5) + (r6 + r7)))

                    acc = lax.fori_loop(0, S // 8, grp,
                                        jnp.zeros((_LANES,), jnp.float32))
                    outb_v[it * CB + bi, pl.ds(lo, _LANES)] = acc * inv
            return carry

        lax.fori_loop(0, NIT, chunk_body, 0)
        pltpu.sync_copy(outb_v, out_hbm.at[pl.ds(base_b, BPW)])

    return pool(table4, x1d)


def _cls_matmul(pooled, wpt, bvec):
    B, D = pooled.shape

    def body(p_ref, w_ref, b_ref, out_ref):
        out_ref[...] = (jnp.dot(p_ref[...], w_ref[...],
                                preferred_element_type=jnp.float32)
                        + b_ref[...])

    return pl.pallas_call(
        body,
        in_specs=[
            pl.BlockSpec((B, D), lambda: (0, 0)),
            pl.BlockSpec((D, _LANES), lambda: (0, 0)),
            pl.BlockSpec((1, _LANES), lambda: (0, 0)),
        ],
        out_specs=pl.BlockSpec((B, _LANES), lambda: (0, 0)),
        out_shape=jax.ShapeDtypeStruct((B, _LANES), jnp.float32),
    )(pooled, wpt, bvec)


def kernel(x, table, W, b):
    B, S = x.shape
    V, D = table.shape
    C = W.shape[0]
    wpt = jnp.zeros((D, _LANES), jnp.float32).at[:, :C].set(W.T)
    bvec = jnp.zeros((1, _LANES), jnp.float32).at[0, :C].set(b)
    table4 = _fmt(jnp.swapaxes(table, 0, 1)).reshape(4 * V, D)
    pooled = _sc_pool(table4, x.astype(jnp.int32).reshape(B * S), B, S)
    logits = _cls_matmul(pooled, wpt, bvec)
    return logits[:, :C]
